# Initial kernel scaffold; baseline (speedup 1.0000x reference)
import jax
import jax.numpy as jnp
from jax.experimental import pallas as pl


def kernel(features, edge_index, W_front, b_front, W_g1, b_g1, W_g2, b_g2, W_b1, b_b1, W_b2, b_b2):
    raise NotImplementedError("write your pallas kernel here")


# speedup vs baseline: 8.5934x; 8.5934x over previous
import functools

import jax
import jax.numpy as jnp
from jax import lax
from jax.experimental import pallas as pl
from jax.experimental.pallas import tpu as pltpu
from jax.experimental.pallas import tpu_sc as plsc

NN = 10000
DD = 128
EE = 320000
NC = 2
NS = 16
CHUNK = 128
EPC = EE // NC
NCHUNKS = EPC // CHUNK
FULL = NCHUNKS // NS
EXTRA = NCHUNKS - FULL * NS
RPT = 624
REM = NN - RPT * NS
DCOL = 64

_sc_mesh = plsc.VectorSubcoreMesh(core_axis_name="c", subcore_axis_name="s")


def _zero_acc_rows(buf, acc, s):
    for k in range(4):
        pltpu.sync_copy(buf, acc.at[pl.ds(s * RPT + k * CHUNK, CHUNK)])
    pltpu.sync_copy(buf.at[pl.ds(0, RPT - 4 * CHUNK)],
                    acc.at[pl.ds(s * RPT + 4 * CHUNK, RPT - 4 * CHUNK)])

    @pl.when(s == NS - 1)
    def _():
        pltpu.sync_copy(buf.at[pl.ds(0, REM)], acc.at[pl.ds(RPT * NS, REM)])


def _drain_acc(acc, p0_h, p1_h, c, s):
    rr = pl.ds(s * RPT, RPT)
    tl = pl.ds(RPT * NS, REM)

    @pl.when(c == 0)
    def _():
        pltpu.sync_copy(acc.at[rr], p0_h.at[rr])

        @pl.when(s == NS - 1)
        def _():
            pltpu.sync_copy(acc.at[tl], p0_h.at[tl])

    @pl.when(c == 1)
    def _():
        pltpu.sync_copy(acc.at[rr], p1_h.at[rr])

        @pl.when(s == NS - 1)
        def _():
            pltpu.sync_copy(acc.at[tl], p1_h.at[tl])


@functools.partial(
    pl.kernel,
    out_type=[jax.ShapeDtypeStruct((NN, DD), jnp.float32)] * 2,
    mesh=_sc_mesh,
    scratch_types=[
        pltpu.VMEM((CHUNK,), jnp.int32),
        pltpu.VMEM((CHUNK,), jnp.int32),
        pltpu.VMEM((CHUNK, DD), jnp.float32),
        pltpu.VMEM((CHUNK, DD), jnp.float32),
        pltpu.VMEM_SHARED((NN, DD), jnp.float32),
    ],
)
def _deg_call(src_h, dst_h, d0_h, d1_h, sidx, didx, e0, e1, acc):
    c = lax.axis_index("c")
    s = lax.axis_index("s")
    ebase = c * EPC
    first = jnp.where(lax.iota(jnp.int32, 16) == 0, 1.0, 0.0)
    zeros16 = jnp.zeros((16,), jnp.float32)

    def fz(t, carry):
        e0[t // 8, pl.ds((t % 8) * 16, 16)] = zeros16
        return carry
    lax.fori_loop(0, CHUNK * 8, fz, 0)
    _zero_acc_rows(e0, acc, s)

    def fe(i, carry):
        e0[i, pl.ds(0, 16)] = first
        for j in range(8):
            e1[i, pl.ds(j * 16, 16)] = first if j * 16 == DCOL else zeros16
        return carry
    lax.fori_loop(0, CHUNK, fe, 0)
    plsc.subcore_barrier()

    def chunk_op(g):
        off = ebase + g * CHUNK
        pltpu.sync_copy(src_h.at[pl.ds(off, CHUNK)], sidx)
        pltpu.sync_copy(dst_h.at[pl.ds(off, CHUNK)], didx)
        pltpu.sync_copy(e0, acc.at[sidx], add=True)
        pltpu.sync_copy(e1, acc.at[didx], add=True)

    def body(i, carry):
        chunk_op(i * NS + s)
        return carry
    lax.fori_loop(0, FULL, body, 0)

    @pl.when(s < EXTRA)
    def _():
        chunk_op(FULL * NS + s)

    plsc.subcore_barrier()
    _drain_acc(acc, d0_h, d1_h, c, s)


@functools.partial(
    pl.kernel,
    out_type=[jax.ShapeDtypeStruct((NN, DD), jnp.float32)] * 2,
    mesh=_sc_mesh,
    scratch_types=[
        pltpu.VMEM((CHUNK,), jnp.int32),
        pltpu.VMEM((CHUNK,), jnp.int32),
        pltpu.VMEM((CHUNK, DD), jnp.float32),
        pltpu.VMEM_SHARED((NN, DD), jnp.float32),
    ],
)
def _agg_call(src_h, dst_h, t_h, p0_h, p1_h, sidx, didx, rows, acc):
    c = lax.axis_index("c")
    s = lax.axis_index("s")
    ebase = c * EPC

    def fill_zero(t, carry):
        rows[t // 8, pl.ds((t % 8) * 16, 16)] = jnp.zeros((16,), jnp.float32)
        return carry
    lax.fori_loop(0, CHUNK * 8, fill_zero, 0)
    _zero_acc_rows(rows, acc, s)
    plsc.subcore_barrier()

    def chunk_op(g):
        off = ebase + g * CHUNK
        pltpu.sync_copy(src_h.at[pl.ds(off, CHUNK)], sidx)
        pltpu.sync_copy(dst_h.at[pl.ds(off, CHUNK)], didx)
        pltpu.sync_copy(t_h.at[sidx], rows)
        pltpu.sync_copy(rows, acc.at[didx], add=True)

    def body(i, carry):
        chunk_op(i * NS + s)
        return carry
    lax.fori_loop(0, FULL, body, 0)

    @pl.when(s < EXTRA)
    def _():
        chunk_op(FULL * NS + s)

    plsc.subcore_barrier()
    _drain_acc(acc, p0_h, p1_h, c, s)


_BLK = 1000
_GRID = NN // _BLK


def _norm(d0, d1, col):
    deg = d0[:, col] + d1[:, col]
    return jnp.where(deg > 0.0, lax.rsqrt(jnp.maximum(deg, 1.0)), 0.0)


def _front_body(x_ref, d0_ref, d1_ref, wf_ref, bf_ref, wg1_ref, o_ref):
    h0 = jnp.dot(x_ref[...], wf_ref[...],
                 preferred_element_type=jnp.float32) + bf_ref[...]
    ns = _norm(d0_ref[...], d1_ref[...], 0)
    o_ref[...] = jnp.dot(h0, wg1_ref[...],
                         preferred_element_type=jnp.float32) * ns[:, None]


def _mid_body(p0_ref, p1_ref, d0_ref, d1_ref, bg1_ref, wg2_ref, o_ref):
    agg = p0_ref[...] + p1_ref[...]
    nd = _norm(d0_ref[...], d1_ref[...], DCOL)
    h1 = jnp.maximum(agg * nd[:, None] + bg1_ref[...], 0.0)
    ns = _norm(d0_ref[...], d1_ref[...], 0)
    o_ref[...] = jnp.dot(h1, wg2_ref[...],
                         preferred_element_type=jnp.float32) * ns[:, None]


def _back_body(p0_ref, p1_ref, d0_ref, d1_ref, bg2_ref,
               wb1_ref, bb1_ref, wb2_ref, bb2_ref, o_ref):
    agg = p0_ref[...] + p1_ref[...]
    nd = _norm(d0_ref[...], d1_ref[...], DCOL)
    h2 = agg * nd[:, None] + bg2_ref[...]
    h3 = jnp.dot(h2, wb1_ref[...],
                 preferred_element_type=jnp.float32) + bb1_ref[...]
    o_ref[...] = jnp.dot(h3, wb2_ref[...],
                         preferred_element_type=jnp.float32) + bb2_ref[...]


def _row_spec(w):
    return pl.BlockSpec((_BLK, w), lambda i: (i, 0))


def _full_spec(h, w):
    return pl.BlockSpec((h, w), lambda i: (0, 0))


_front_call = pl.pallas_call(
    _front_body,
    grid=(_GRID,),
    in_specs=[_row_spec(DD), _row_spec(DD), _row_spec(DD),
              _full_spec(DD, DD), _full_spec(1, DD), _full_spec(DD, DD)],
    out_specs=_row_spec(DD),
    out_shape=jax.ShapeDtypeStruct((NN, DD), jnp.float32),
)

_mid_call = pl.pallas_call(
    _mid_body,
    grid=(_GRID,),
    in_specs=[_row_spec(DD), _row_spec(DD), _row_spec(DD), _row_spec(DD),
              _full_spec(1, DD), _full_spec(DD, DD)],
    out_specs=_row_spec(DD),
    out_shape=jax.ShapeDtypeStruct((NN, DD), jnp.float32),
)

_back_call = pl.pallas_call(
    _back_body,
    grid=(_GRID,),
    in_specs=[_row_spec(DD), _row_spec(DD), _row_spec(DD), _row_spec(DD),
              _full_spec(1, DD), _full_spec(DD, 64), _full_spec(1, 64),
              _full_spec(64, 1), _full_spec(1, 1)],
    out_specs=pl.BlockSpec((_BLK, 1), lambda i: (i, 0)),
    out_shape=jax.ShapeDtypeStruct((NN, 1), jnp.float32),
)


def kernel(features, edge_index, W_front, b_front, W_g1, b_g1, W_g2, b_g2,
           W_b1, b_b1, W_b2, b_b2):
    src = edge_index[0]
    dst = edge_index[1]
    d0, d1 = _deg_call(src, dst)
    t1 = _front_call(features, d0, d1, W_front,
                     b_front.reshape(1, DD), W_g1)
    p0, p1 = _agg_call(src, dst, t1)
    t2 = _mid_call(p0, p1, d0, d1, b_g1.reshape(1, DD), W_g2)
    q0, q1 = _agg_call(src, dst, t2)
    out = _back_call(q0, q1, d0, d1, b_g2.reshape(1, DD),
                     W_b1, b_b1.reshape(1, 64), W_b2, b_b2.reshape(1, 1))
    return out.reshape(-1)


# speedup vs baseline: 11.4902x; 1.3371x over previous
import functools

import jax
import jax.numpy as jnp
from jax import lax
from jax.experimental import pallas as pl
from jax.experimental.pallas import tpu as pltpu
from jax.experimental.pallas import tpu_sc as plsc

NN = 10000
DD = 128
EE = 320000
NC = 2
NS = 16
CHUNK = 128
EPC = EE // NC
NCHUNKS = EPC // CHUNK
FULL = NCHUNKS // NS
EXTRA = NCHUNKS - FULL * NS
RPT = 624
REM = NN - RPT * NS
DCOL = 64

_sc_mesh = plsc.VectorSubcoreMesh(core_axis_name="c", subcore_axis_name="s")


def _zero_acc_rows(buf, acc, s):
    for k in range(4):
        pltpu.sync_copy(buf, acc.at[pl.ds(s * RPT + k * CHUNK, CHUNK)])
    pltpu.sync_copy(buf.at[pl.ds(0, RPT - 4 * CHUNK)],
                    acc.at[pl.ds(s * RPT + 4 * CHUNK, RPT - 4 * CHUNK)])

    @pl.when(s == NS - 1)
    def _():
        pltpu.sync_copy(buf.at[pl.ds(0, REM)], acc.at[pl.ds(RPT * NS, REM)])


def _drain_acc(acc, p0_h, p1_h, c, s):
    rr = pl.ds(s * RPT, RPT)
    tl = pl.ds(RPT * NS, REM)

    @pl.when(c == 0)
    def _():
        pltpu.sync_copy(acc.at[rr], p0_h.at[rr])

        @pl.when(s == NS - 1)
        def _():
            pltpu.sync_copy(acc.at[tl], p0_h.at[tl])

    @pl.when(c == 1)
    def _():
        pltpu.sync_copy(acc.at[rr], p1_h.at[rr])

        @pl.when(s == NS - 1)
        def _():
            pltpu.sync_copy(acc.at[tl], p1_h.at[tl])


@functools.partial(
    pl.kernel,
    out_type=[jax.ShapeDtypeStruct((NN, DD), jnp.float32)] * 2,
    mesh=_sc_mesh,
    scratch_types=[
        pltpu.VMEM((CHUNK,), jnp.int32),
        pltpu.VMEM((CHUNK,), jnp.int32),
        pltpu.VMEM((CHUNK, DD), jnp.float32),
        pltpu.VMEM((CHUNK, DD), jnp.float32),
        pltpu.VMEM_SHARED((NN, DD), jnp.float32),
    ],
)
def _deg_call(src_h, dst_h, d0_h, d1_h, sidx, didx, e0, e1, acc):
    c = lax.axis_index("c")
    s = lax.axis_index("s")
    ebase = c * EPC
    first = jnp.where(lax.iota(jnp.int32, 16) == 0, 1.0, 0.0)
    zeros16 = jnp.zeros((16,), jnp.float32)

    def fz(t, carry):
        e0[t // 8, pl.ds((t % 8) * 16, 16)] = zeros16
        return carry
    lax.fori_loop(0, CHUNK * 8, fz, 0)
    _zero_acc_rows(e0, acc, s)

    def fe(i, carry):
        e0[i, pl.ds(0, 16)] = first
        for j in range(8):
            e1[i, pl.ds(j * 16, 16)] = first if j * 16 == DCOL else zeros16
        return carry
    lax.fori_loop(0, CHUNK, fe, 0)
    plsc.subcore_barrier()

    def chunk_op(g):
        off = ebase + g * CHUNK
        pltpu.sync_copy(src_h.at[pl.ds(off, CHUNK)], sidx)
        pltpu.sync_copy(dst_h.at[pl.ds(off, CHUNK)], didx)
        pltpu.sync_copy(e0, acc.at[sidx], add=True)
        pltpu.sync_copy(e1, acc.at[didx], add=True)

    def body(i, carry):
        chunk_op(i * NS + s)
        return carry
    lax.fori_loop(0, FULL, body, 0)

    @pl.when(s < EXTRA)
    def _():
        chunk_op(FULL * NS + s)

    plsc.subcore_barrier()
    _drain_acc(acc, d0_h, d1_h, c, s)


@functools.partial(
    pl.kernel,
    out_type=[jax.ShapeDtypeStruct((NN, DD), jnp.float32)] * 2,
    mesh=_sc_mesh,
    scratch_types=[
        pltpu.VMEM((CHUNK,), jnp.int32),
        pltpu.VMEM((CHUNK,), jnp.int32),
        pltpu.VMEM((CHUNK,), jnp.int32),
        pltpu.VMEM((CHUNK,), jnp.int32),
        pltpu.VMEM((CHUNK, DD), jnp.float32),
        pltpu.VMEM((CHUNK, DD), jnp.float32),
        pltpu.SemaphoreType.DMA,
        pltpu.SemaphoreType.DMA,
        pltpu.VMEM_SHARED((NN, DD), jnp.float32),
    ],
)
def _agg_call(src_h, dst_h, t_h, p0_h, p1_h,
              sidx0, didx0, sidx1, didx1, rows0, rows1, sem0, sem1, acc):
    c = lax.axis_index("c")
    s = lax.axis_index("s")
    ebase = c * EPC

    def fill_zero(t, carry):
        rows0[t // 8, pl.ds((t % 8) * 16, 16)] = jnp.zeros((16,), jnp.float32)
        return carry
    lax.fori_loop(0, CHUNK * 8, fill_zero, 0)
    _zero_acc_rows(rows0, acc, s)
    plsc.subcore_barrier()

    def issue(g, sidx, didx, rows, sem):
        off = ebase + g * CHUNK
        pltpu.sync_copy(src_h.at[pl.ds(off, CHUNK)], sidx)
        pltpu.sync_copy(dst_h.at[pl.ds(off, CHUNK)], didx)
        pltpu.async_copy(t_h.at[sidx], rows, sem)

    def drain(sidx, didx, rows, sem):
        pltpu.make_async_copy(t_h.at[sidx], rows, sem).wait()
        pltpu.sync_copy(rows, acc.at[didx], add=True)

    issue(s, sidx0, didx0, rows0, sem0)

    def body(k, carry):
        g = (2 * k) * NS + s
        issue(g + NS, sidx1, didx1, rows1, sem1)
        drain(sidx0, didx0, rows0, sem0)

        @pl.when(k < FULL // 2 - 1)
        def _():
            issue(g + 2 * NS, sidx0, didx0, rows0, sem0)
        drain(sidx1, didx1, rows1, sem1)
        return carry
    lax.fori_loop(0, FULL // 2, body, 0)

    @pl.when(s < EXTRA)
    def _():
        issue(FULL * NS + s, sidx0, didx0, rows0, sem0)
        drain(sidx0, didx0, rows0, sem0)

    plsc.subcore_barrier()
    _drain_acc(acc, p0_h, p1_h, c, s)


_BLK = 1000
_GRID = NN // _BLK


def _norm(d0, d1, col):
    deg = d0[:, col] + d1[:, col]
    return jnp.where(deg > 0.0, lax.rsqrt(jnp.maximum(deg, 1.0)), 0.0)


def _front_body(x_ref, d0_ref, d1_ref, wf_ref, bf_ref, wg1_ref, o_ref):
    h0 = jnp.dot(x_ref[...], wf_ref[...],
                 preferred_element_type=jnp.float32) + bf_ref[...]
    ns = _norm(d0_ref[...], d1_ref[...], 0)
    o_ref[...] = jnp.dot(h0, wg1_ref[...],
                         preferred_element_type=jnp.float32) * ns[:, None]


def _mid_body(p0_ref, p1_ref, d0_ref, d1_ref, bg1_ref, wg2_ref, o_ref):
    agg = p0_ref[...] + p1_ref[...]
    nd = _norm(d0_ref[...], d1_ref[...], DCOL)
    h1 = jnp.maximum(agg * nd[:, None] + bg1_ref[...], 0.0)
    ns = _norm(d0_ref[...], d1_ref[...], 0)
    o_ref[...] = jnp.dot(h1, wg2_ref[...],
                         preferred_element_type=jnp.float32) * ns[:, None]


def _back_body(p0_ref, p1_ref, d0_ref, d1_ref, bg2_ref,
               wb1_ref, bb1_ref, wb2_ref, bb2_ref, o_ref):
    agg = p0_ref[...] + p1_ref[...]
    nd = _norm(d0_ref[...], d1_ref[...], DCOL)
    h2 = agg * nd[:, None] + bg2_ref[...]
    h3 = jnp.dot(h2, wb1_ref[...],
                 preferred_element_type=jnp.float32) + bb1_ref[...]
    o_ref[...] = jnp.dot(h3, wb2_ref[...],
                         preferred_element_type=jnp.float32) + bb2_ref[...]


def _row_spec(w):
    return pl.BlockSpec((_BLK, w), lambda i: (i, 0))


def _full_spec(h, w):
    return pl.BlockSpec((h, w), lambda i: (0, 0))


_front_call = pl.pallas_call(
    _front_body,
    grid=(_GRID,),
    in_specs=[_row_spec(DD), _row_spec(DD), _row_spec(DD),
              _full_spec(DD, DD), _full_spec(1, DD), _full_spec(DD, DD)],
    out_specs=_row_spec(DD),
    out_shape=jax.ShapeDtypeStruct((NN, DD), jnp.float32),
)

_mid_call = pl.pallas_call(
    _mid_body,
    grid=(_GRID,),
    in_specs=[_row_spec(DD), _row_spec(DD), _row_spec(DD), _row_spec(DD),
              _full_spec(1, DD), _full_spec(DD, DD)],
    out_specs=_row_spec(DD),
    out_shape=jax.ShapeDtypeStruct((NN, DD), jnp.float32),
)

_back_call = pl.pallas_call(
    _back_body,
    grid=(_GRID,),
    in_specs=[_row_spec(DD), _row_spec(DD), _row_spec(DD), _row_spec(DD),
              _full_spec(1, DD), _full_spec(DD, 64), _full_spec(1, 64),
              _full_spec(64, 1), _full_spec(1, 1)],
    out_specs=pl.BlockSpec((_BLK, 1), lambda i: (i, 0)),
    out_shape=jax.ShapeDtypeStruct((NN, 1), jnp.float32),
)


def kernel(features, edge_index, W_front, b_front, W_g1, b_g1, W_g2, b_g2,
           W_b1, b_b1, W_b2, b_b2):
    src = edge_index[0]
    dst = edge_index[1]
    d0, d1 = _deg_call(src, dst)
    t1 = _front_call(features, d0, d1, W_front,
                     b_front.reshape(1, DD), W_g1)
    p0, p1 = _agg_call(src, dst, t1)
    t2 = _mid_call(p0, p1, d0, d1, b_g1.reshape(1, DD), W_g2)
    q0, q1 = _agg_call(src, dst, t2)
    out = _back_call(q0, q1, d0, d1, b_g2.reshape(1, DD),
                     W_b1, b_b1.reshape(1, 64), W_b2, b_b2.reshape(1, 1))
    return out.reshape(-1)


# speedup vs baseline: 13.0095x; 1.1322x over previous
import functools

import jax
import jax.numpy as jnp
from jax import lax
from jax.experimental import pallas as pl
from jax.experimental.pallas import tpu as pltpu
from jax.experimental.pallas import tpu_sc as plsc

NN = 10000
DD = 128
EE = 320000
NC = 2
NS = 16
CHUNK = 128
EPC = EE // NC
NCHUNKS = EPC // CHUNK
FULL = NCHUNKS // NS
EXTRA = NCHUNKS - FULL * NS
RPT = 624
REM = NN - RPT * NS
DCOL = 64

_sc_mesh = plsc.VectorSubcoreMesh(core_axis_name="c", subcore_axis_name="s")


def _zero_acc_rows(buf, acc, s):
    for k in range(4):
        pltpu.sync_copy(buf, acc.at[pl.ds(s * RPT + k * CHUNK, CHUNK)])
    pltpu.sync_copy(buf.at[pl.ds(0, RPT - 4 * CHUNK)],
                    acc.at[pl.ds(s * RPT + 4 * CHUNK, RPT - 4 * CHUNK)])

    @pl.when(s == NS - 1)
    def _():
        pltpu.sync_copy(buf.at[pl.ds(0, REM)], acc.at[pl.ds(RPT * NS, REM)])


def _drain_acc(acc, p0_h, p1_h, c, s):
    rr = pl.ds(s * RPT, RPT)
    tl = pl.ds(RPT * NS, REM)

    @pl.when(c == 0)
    def _():
        pltpu.sync_copy(acc.at[rr], p0_h.at[rr])

        @pl.when(s == NS - 1)
        def _():
            pltpu.sync_copy(acc.at[tl], p0_h.at[tl])

    @pl.when(c == 1)
    def _():
        pltpu.sync_copy(acc.at[rr], p1_h.at[rr])

        @pl.when(s == NS - 1)
        def _():
            pltpu.sync_copy(acc.at[tl], p1_h.at[tl])


@functools.partial(
    pl.kernel,
    out_type=[jax.ShapeDtypeStruct((NN, DD), jnp.float32)] * 2,
    mesh=_sc_mesh,
    scratch_types=[
        pltpu.VMEM((CHUNK,), jnp.int32),
        pltpu.VMEM((CHUNK,), jnp.int32),
        pltpu.VMEM((CHUNK,), jnp.int32),
        pltpu.VMEM((CHUNK,), jnp.int32),
        pltpu.SemaphoreType.DMA,
        pltpu.SemaphoreType.DMA,
        pltpu.VMEM((CHUNK, DD), jnp.float32),
        pltpu.VMEM((CHUNK, DD), jnp.float32),
        pltpu.VMEM_SHARED((NN, DD), jnp.float32),
    ],
)
def _deg_call(src_h, dst_h, d0_h, d1_h,
              sidx0, didx0, sidx1, didx1, sem0, sem1, e0, e1, acc):
    c = lax.axis_index("c")
    s = lax.axis_index("s")
    ebase = c * EPC
    first = jnp.where(lax.iota(jnp.int32, 16) == 0, 1.0, 0.0)
    zeros16 = jnp.zeros((16,), jnp.float32)

    def fz(t, carry):
        e0[t // 8, pl.ds((t % 8) * 16, 16)] = zeros16
        return carry
    lax.fori_loop(0, CHUNK * 8, fz, 0)
    _zero_acc_rows(e0, acc, s)

    def fe(i, carry):
        e0[i, pl.ds(0, 16)] = first
        for j in range(8):
            e1[i, pl.ds(j * 16, 16)] = first if j * 16 == DCOL else zeros16
        return carry
    lax.fori_loop(0, CHUNK, fe, 0)
    plsc.subcore_barrier()

    def issue(g, sidx, didx, sem):
        off = ebase + g * CHUNK
        pltpu.async_copy(src_h.at[pl.ds(off, CHUNK)], sidx, sem)
        pltpu.async_copy(dst_h.at[pl.ds(off, CHUNK)], didx, sem)

    def drain(g, sidx, didx, sem):
        off = ebase + g * CHUNK
        pltpu.make_async_copy(src_h.at[pl.ds(off, CHUNK)], sidx, sem).wait()
        pltpu.make_async_copy(dst_h.at[pl.ds(off, CHUNK)], didx, sem).wait()
        pltpu.sync_copy(e0, acc.at[sidx], add=True)
        pltpu.sync_copy(e1, acc.at[didx], add=True)

    issue(s, sidx0, didx0, sem0)

    def body(k, carry):
        g = (2 * k) * NS + s
        issue(g + NS, sidx1, didx1, sem1)
        drain(g, sidx0, didx0, sem0)

        @pl.when(k < FULL // 2 - 1)
        def _():
            issue(g + 2 * NS, sidx0, didx0, sem0)
        drain(g + NS, sidx1, didx1, sem1)
        return carry
    lax.fori_loop(0, FULL // 2, body, 0)

    @pl.when(s < EXTRA)
    def _():
        issue(FULL * NS + s, sidx0, didx0, sem0)
        drain(FULL * NS + s, sidx0, didx0, sem0)

    plsc.subcore_barrier()
    _drain_acc(acc, d0_h, d1_h, c, s)


@functools.partial(
    pl.kernel,
    out_type=[jax.ShapeDtypeStruct((NN, DD), jnp.float32)] * 2,
    mesh=_sc_mesh,
    scratch_types=[
        pltpu.VMEM((CHUNK,), jnp.int32),
        pltpu.VMEM((CHUNK,), jnp.int32),
        pltpu.VMEM((CHUNK,), jnp.int32),
        pltpu.VMEM((CHUNK,), jnp.int32),
        pltpu.VMEM((CHUNK, DD), jnp.float32),
        pltpu.VMEM((CHUNK, DD), jnp.float32),
        pltpu.SemaphoreType.DMA,
        pltpu.SemaphoreType.DMA,
        pltpu.VMEM_SHARED((NN, DD), jnp.float32),
    ],
)
def _agg_call(src_h, dst_h, t_h, p0_h, p1_h,
              sidx0, didx0, sidx1, didx1, rows0, rows1, sem0, sem1, acc):
    c = lax.axis_index("c")
    s = lax.axis_index("s")
    ebase = c * EPC

    def fill_zero(t, carry):
        rows0[t // 8, pl.ds((t % 8) * 16, 16)] = jnp.zeros((16,), jnp.float32)
        return carry
    lax.fori_loop(0, CHUNK * 8, fill_zero, 0)
    _zero_acc_rows(rows0, acc, s)
    plsc.subcore_barrier()

    def issue(g, sidx, didx, rows, sem):
        off = ebase + g * CHUNK
        pltpu.sync_copy(src_h.at[pl.ds(off, CHUNK)], sidx)
        pltpu.sync_copy(dst_h.at[pl.ds(off, CHUNK)], didx)
        pltpu.async_copy(t_h.at[sidx], rows, sem)

    def drain(sidx, didx, rows, sem):
        pltpu.make_async_copy(t_h.at[sidx], rows, sem).wait()
        pltpu.sync_copy(rows, acc.at[didx], add=True)

    issue(s, sidx0, didx0, rows0, sem0)

    def body(k, carry):
        g = (2 * k) * NS + s
        issue(g + NS, sidx1, didx1, rows1, sem1)
        drain(sidx0, didx0, rows0, sem0)

        @pl.when(k < FULL // 2 - 1)
        def _():
            issue(g + 2 * NS, sidx0, didx0, rows0, sem0)
        drain(sidx1, didx1, rows1, sem1)
        return carry
    lax.fori_loop(0, FULL // 2, body, 0)

    @pl.when(s < EXTRA)
    def _():
        issue(FULL * NS + s, sidx0, didx0, rows0, sem0)
        drain(sidx0, didx0, rows0, sem0)

    plsc.subcore_barrier()
    _drain_acc(acc, p0_h, p1_h, c, s)


_BLK = 1000
_GRID = NN // _BLK


def _norm(d0, d1, col):
    deg = d0[:, col] + d1[:, col]
    return jnp.where(deg > 0.0, lax.rsqrt(jnp.maximum(deg, 1.0)), 0.0)


def _front_body(x_ref, d0_ref, d1_ref, wf_ref, bf_ref, wg1_ref, o_ref):
    h0 = jnp.dot(x_ref[...], wf_ref[...],
                 preferred_element_type=jnp.float32) + bf_ref[...]
    ns = _norm(d0_ref[...], d1_ref[...], 0)
    o_ref[...] = jnp.dot(h0, wg1_ref[...],
                         preferred_element_type=jnp.float32) * ns[:, None]


def _mid_body(p0_ref, p1_ref, d0_ref, d1_ref, bg1_ref, wg2_ref, o_ref):
    agg = p0_ref[...] + p1_ref[...]
    nd = _norm(d0_ref[...], d1_ref[...], DCOL)
    h1 = jnp.maximum(agg * nd[:, None] + bg1_ref[...], 0.0)
    ns = _norm(d0_ref[...], d1_ref[...], 0)
    o_ref[...] = jnp.dot(h1, wg2_ref[...],
                         preferred_element_type=jnp.float32) * ns[:, None]


def _back_body(p0_ref, p1_ref, d0_ref, d1_ref, bg2_ref,
               wb1_ref, bb1_ref, wb2_ref, bb2_ref, o_ref):
    agg = p0_ref[...] + p1_ref[...]
    nd = _norm(d0_ref[...], d1_ref[...], DCOL)
    h2 = agg * nd[:, None] + bg2_ref[...]
    h3 = jnp.dot(h2, wb1_ref[...],
                 preferred_element_type=jnp.float32) + bb1_ref[...]
    o_ref[...] = jnp.dot(h3, wb2_ref[...],
                         preferred_element_type=jnp.float32) + bb2_ref[...]


def _row_spec(w):
    return pl.BlockSpec((_BLK, w), lambda i: (i, 0))


def _full_spec(h, w):
    return pl.BlockSpec((h, w), lambda i: (0, 0))


_front_call = pl.pallas_call(
    _front_body,
    grid=(_GRID,),
    in_specs=[_row_spec(DD), _row_spec(DD), _row_spec(DD),
              _full_spec(DD, DD), _full_spec(1, DD), _full_spec(DD, DD)],
    out_specs=_row_spec(DD),
    out_shape=jax.ShapeDtypeStruct((NN, DD), jnp.float32),
)

_mid_call = pl.pallas_call(
    _mid_body,
    grid=(_GRID,),
    in_specs=[_row_spec(DD), _row_spec(DD), _row_spec(DD), _row_spec(DD),
              _full_spec(1, DD), _full_spec(DD, DD)],
    out_specs=_row_spec(DD),
    out_shape=jax.ShapeDtypeStruct((NN, DD), jnp.float32),
)

_back_call = pl.pallas_call(
    _back_body,
    grid=(_GRID,),
    in_specs=[_row_spec(DD), _row_spec(DD), _row_spec(DD), _row_spec(DD),
              _full_spec(1, DD), _full_spec(DD, 64), _full_spec(1, 64),
              _full_spec(64, 1), _full_spec(1, 1)],
    out_specs=pl.BlockSpec((_BLK, 1), lambda i: (i, 0)),
    out_shape=jax.ShapeDtypeStruct((NN, 1), jnp.float32),
)


def kernel(features, edge_index, W_front, b_front, W_g1, b_g1, W_g2, b_g2,
           W_b1, b_b1, W_b2, b_b2):
    src = edge_index[0]
    dst = edge_index[1]
    d0, d1 = _deg_call(src, dst)
    t1 = _front_call(features, d0, d1, W_front,
                     b_front.reshape(1, DD), W_g1)
    p0, p1 = _agg_call(src, dst, t1)
    t2 = _mid_call(p0, p1, d0, d1, b_g1.reshape(1, DD), W_g2)
    q0, q1 = _agg_call(src, dst, t2)
    out = _back_call(q0, q1, d0, d1, b_g2.reshape(1, DD),
                     W_b1, b_b1.reshape(1, 64), W_b2, b_b2.reshape(1, 1))
    return out.reshape(-1)


# speedup vs baseline: 15.6205x; 1.2007x over previous
import functools

import jax
import jax.numpy as jnp
from jax import lax
from jax.experimental import pallas as pl
from jax.experimental.pallas import tpu as pltpu
from jax.experimental.pallas import tpu_sc as plsc

NN = 10000
DD = 128
EE = 320000
NC = 2
NS = 16
CHUNK = 128
EPC = EE // NC
NCHUNKS = EPC // CHUNK
FULL = NCHUNKS // NS
EXTRA = NCHUNKS - FULL * NS
RPT = 624
REM = NN - RPT * NS
DCOL = 64

_sc_mesh = plsc.VectorSubcoreMesh(core_axis_name="c", subcore_axis_name="s")


def _zero_acc_rows(buf, acc, s):
    for k in range(4):
        pltpu.sync_copy(buf, acc.at[pl.ds(s * RPT + k * CHUNK, CHUNK)])
    pltpu.sync_copy(buf.at[pl.ds(0, RPT - 4 * CHUNK)],
                    acc.at[pl.ds(s * RPT + 4 * CHUNK, RPT - 4 * CHUNK)])

    @pl.when(s == NS - 1)
    def _():
        pltpu.sync_copy(buf.at[pl.ds(0, REM)], acc.at[pl.ds(RPT * NS, REM)])


def _drain_acc(acc, p0_h, p1_h, c, s):
    rr = pl.ds(s * RPT, RPT)
    tl = pl.ds(RPT * NS, REM)

    @pl.when(c == 0)
    def _():
        pltpu.sync_copy(acc.at[rr], p0_h.at[rr])

        @pl.when(s == NS - 1)
        def _():
            pltpu.sync_copy(acc.at[tl], p0_h.at[tl])

    @pl.when(c == 1)
    def _():
        pltpu.sync_copy(acc.at[rr], p1_h.at[rr])

        @pl.when(s == NS - 1)
        def _():
            pltpu.sync_copy(acc.at[tl], p1_h.at[tl])


@functools.partial(
    pl.kernel,
    out_type=[jax.ShapeDtypeStruct((NN, DD), jnp.float32)] * 2,
    mesh=_sc_mesh,
    scratch_types=[
        pltpu.VMEM((CHUNK,), jnp.int32),
        pltpu.VMEM((CHUNK,), jnp.int32),
        pltpu.VMEM((CHUNK,), jnp.int32),
        pltpu.VMEM((CHUNK,), jnp.int32),
        pltpu.SemaphoreType.DMA,
        pltpu.SemaphoreType.DMA,
        pltpu.VMEM((CHUNK, DD), jnp.float32),
        pltpu.VMEM((CHUNK, DD), jnp.float32),
        pltpu.VMEM_SHARED((NN, DD), jnp.float32),
    ],
)
def _deg_call(src_h, dst_h, d0_h, d1_h,
              sidx0, didx0, sidx1, didx1, sem0, sem1, e0, e1, acc):
    c = lax.axis_index("c")
    s = lax.axis_index("s")
    ebase = c * EPC
    first = jnp.where(lax.iota(jnp.int32, 16) == 0, 1.0, 0.0)
    zeros16 = jnp.zeros((16,), jnp.float32)

    def fz(t, carry):
        e0[t // 8, pl.ds((t % 8) * 16, 16)] = zeros16
        return carry
    lax.fori_loop(0, CHUNK * 8, fz, 0)
    _zero_acc_rows(e0, acc, s)

    def fe(i, carry):
        e0[i, pl.ds(0, 16)] = first
        for j in range(8):
            e1[i, pl.ds(j * 16, 16)] = first if j * 16 == DCOL else zeros16
        return carry
    lax.fori_loop(0, CHUNK, fe, 0)
    plsc.subcore_barrier()

    def issue(g, sidx, didx, sem):
        off = ebase + g * CHUNK
        pltpu.async_copy(src_h.at[pl.ds(off, CHUNK)], sidx, sem)
        pltpu.async_copy(dst_h.at[pl.ds(off, CHUNK)], didx, sem)

    def drain(g, sidx, didx, sem):
        off = ebase + g * CHUNK
        pltpu.make_async_copy(src_h.at[pl.ds(off, CHUNK)], sidx, sem).wait()
        pltpu.make_async_copy(dst_h.at[pl.ds(off, CHUNK)], didx, sem).wait()
        pltpu.sync_copy(e0, acc.at[sidx], add=True)
        pltpu.sync_copy(e1, acc.at[didx], add=True)

    issue(s, sidx0, didx0, sem0)

    def body(k, carry):
        g = (2 * k) * NS + s
        issue(g + NS, sidx1, didx1, sem1)
        drain(g, sidx0, didx0, sem0)

        @pl.when(k < FULL // 2 - 1)
        def _():
            issue(g + 2 * NS, sidx0, didx0, sem0)
        drain(g + NS, sidx1, didx1, sem1)
        return carry
    lax.fori_loop(0, FULL // 2, body, 0)

    @pl.when(s < EXTRA)
    def _():
        issue(FULL * NS + s, sidx0, didx0, sem0)
        drain(FULL * NS + s, sidx0, didx0, sem0)

    plsc.subcore_barrier()
    _drain_acc(acc, d0_h, d1_h, c, s)


NSLOT = 6
ROUNDS = FULL // NSLOT


@functools.partial(
    pl.kernel,
    out_type=[jax.ShapeDtypeStruct((NN, DD), jnp.float32)] * 2,
    mesh=_sc_mesh,
    scratch_types=(
        [pltpu.VMEM((CHUNK,), jnp.int32)] * (2 * NSLOT)
        + [pltpu.SemaphoreType.DMA] * NSLOT
        + [
            pltpu.VMEM((CHUNK, DD), jnp.float32),
            pltpu.VMEM((CHUNK, DD), jnp.float32),
            pltpu.SemaphoreType.DMA,
            pltpu.SemaphoreType.DMA,
            pltpu.VMEM_SHARED((NN, DD), jnp.float32),
        ]
    ),
)
def _agg_call(src_h, dst_h, t_h, p0_h, p1_h, *refs):
    sx = refs[0:2 * NSLOT:2]
    dx = refs[1:2 * NSLOT:2]
    isem = refs[2 * NSLOT:3 * NSLOT]
    rows = refs[3 * NSLOT:3 * NSLOT + 2]
    gsem = refs[3 * NSLOT + 2:3 * NSLOT + 4]
    acc = refs[3 * NSLOT + 4]
    c = lax.axis_index("c")
    s = lax.axis_index("s")
    ebase = c * EPC

    def fill_zero(t, carry):
        rows[0][t // 8, pl.ds((t % 8) * 16, 16)] = jnp.zeros((16,), jnp.float32)
        return carry
    lax.fori_loop(0, CHUNK * 8, fill_zero, 0)
    _zero_acc_rows(rows[0], acc, s)
    plsc.subcore_barrier()

    def cidx(j, m):
        return (j * NSLOT + m) * NS + s

    def iload(g, m):
        off = ebase + g * CHUNK
        pltpu.async_copy(src_h.at[pl.ds(off, CHUNK)], sx[m], isem[m])
        pltpu.async_copy(dst_h.at[pl.ds(off, CHUNK)], dx[m], isem[m])

    def iwait(g, m):
        off = ebase + g * CHUNK
        pltpu.make_async_copy(src_h.at[pl.ds(off, CHUNK)], sx[m], isem[m]).wait()
        pltpu.make_async_copy(dst_h.at[pl.ds(off, CHUNK)], dx[m], isem[m]).wait()

    def gather(m, r):
        pltpu.async_copy(t_h.at[sx[m]], rows[r], gsem[r])

    def drain(m, r):
        pltpu.make_async_copy(t_h.at[sx[m]], rows[r], gsem[r]).wait()
        pltpu.sync_copy(rows[r], acc.at[dx[m]], add=True)

    for m in range(NSLOT):
        iload(cidx(0, m), m)
    iwait(cidx(0, 0), 0)
    gather(0, 0)

    def round_body(j, carry):
        for m in range(NSLOT):
            nm = (m + 1) % NSLOT
            if nm != 0:
                iwait(cidx(j, nm), nm)
                gather(nm, nm % 2)
            else:
                @pl.when(j < ROUNDS - 1)
                def _():
                    iwait(cidx(j + 1, 0), 0)
                    gather(0, 0)
            drain(m, m % 2)

            @pl.when(j < ROUNDS - 1)
            def _():
                iload(cidx(j + 1, m), m)
        return carry
    lax.fori_loop(0, ROUNDS, round_body, 0)

    @pl.when(s < EXTRA)
    def _():
        g = FULL * NS + s
        iload(g, 0)
        iwait(g, 0)
        gather(0, 0)
        drain(0, 0)

    plsc.subcore_barrier()
    _drain_acc(acc, p0_h, p1_h, c, s)


_BLK = 1000
_GRID = NN // _BLK


def _norm(d0, d1, col):
    deg = d0[:, col] + d1[:, col]
    return jnp.where(deg > 0.0, lax.rsqrt(jnp.maximum(deg, 1.0)), 0.0)


def _front_body(x_ref, d0_ref, d1_ref, wf_ref, bf_ref, wg1_ref, o_ref):
    h0 = jnp.dot(x_ref[...], wf_ref[...],
                 preferred_element_type=jnp.float32) + bf_ref[...]
    ns = _norm(d0_ref[...], d1_ref[...], 0)
    o_ref[...] = jnp.dot(h0, wg1_ref[...],
                         preferred_element_type=jnp.float32) * ns[:, None]


def _mid_body(p0_ref, p1_ref, d0_ref, d1_ref, bg1_ref, wg2_ref, o_ref):
    agg = p0_ref[...] + p1_ref[...]
    nd = _norm(d0_ref[...], d1_ref[...], DCOL)
    h1 = jnp.maximum(agg * nd[:, None] + bg1_ref[...], 0.0)
    ns = _norm(d0_ref[...], d1_ref[...], 0)
    o_ref[...] = jnp.dot(h1, wg2_ref[...],
                         preferred_element_type=jnp.float32) * ns[:, None]


def _back_body(p0_ref, p1_ref, d0_ref, d1_ref, bg2_ref,
               wb1_ref, bb1_ref, wb2_ref, bb2_ref, o_ref):
    agg = p0_ref[...] + p1_ref[...]
    nd = _norm(d0_ref[...], d1_ref[...], DCOL)
    h2 = agg * nd[:, None] + bg2_ref[...]
    h3 = jnp.dot(h2, wb1_ref[...],
                 preferred_element_type=jnp.float32) + bb1_ref[...]
    o_ref[...] = jnp.dot(h3, wb2_ref[...],
                         preferred_element_type=jnp.float32) + bb2_ref[...]


def _row_spec(w):
    return pl.BlockSpec((_BLK, w), lambda i: (i, 0))


def _full_spec(h, w):
    return pl.BlockSpec((h, w), lambda i: (0, 0))


_front_call = pl.pallas_call(
    _front_body,
    grid=(_GRID,),
    in_specs=[_row_spec(DD), _row_spec(DD), _row_spec(DD),
              _full_spec(DD, DD), _full_spec(1, DD), _full_spec(DD, DD)],
    out_specs=_row_spec(DD),
    out_shape=jax.ShapeDtypeStruct((NN, DD), jnp.float32),
)

_mid_call = pl.pallas_call(
    _mid_body,
    grid=(_GRID,),
    in_specs=[_row_spec(DD), _row_spec(DD), _row_spec(DD), _row_spec(DD),
              _full_spec(1, DD), _full_spec(DD, DD)],
    out_specs=_row_spec(DD),
    out_shape=jax.ShapeDtypeStruct((NN, DD), jnp.float32),
)

_back_call = pl.pallas_call(
    _back_body,
    grid=(_GRID,),
    in_specs=[_row_spec(DD), _row_spec(DD), _row_spec(DD), _row_spec(DD),
              _full_spec(1, DD), _full_spec(DD, 64), _full_spec(1, 64),
              _full_spec(64, 1), _full_spec(1, 1)],
    out_specs=pl.BlockSpec((_BLK, 1), lambda i: (i, 0)),
    out_shape=jax.ShapeDtypeStruct((NN, 1), jnp.float32),
)


def kernel(features, edge_index, W_front, b_front, W_g1, b_g1, W_g2, b_g2,
           W_b1, b_b1, W_b2, b_b2):
    src = edge_index[0]
    dst = edge_index[1]
    d0, d1 = _deg_call(src, dst)
    t1 = _front_call(features, d0, d1, W_front,
                     b_front.reshape(1, DD), W_g1)
    p0, p1 = _agg_call(src, dst, t1)
    t2 = _mid_call(p0, p1, d0, d1, b_g1.reshape(1, DD), W_g2)
    q0, q1 = _agg_call(src, dst, t2)
    out = _back_call(q0, q1, d0, d1, b_g2.reshape(1, DD),
                     W_b1, b_b1.reshape(1, 64), W_b2, b_b2.reshape(1, 1))
    return out.reshape(-1)
